# baseline (speedup 1.0000x reference)
import functools

import jax
import jax.numpy as jnp
from jax import lax
from jax.experimental import pallas as pl
from jax.experimental.pallas import tpu as pltpu
from jax.experimental.pallas import tpu_sc as plsc

_BATCH = 4
_SEQ = 2048
_EMBED = 64
_FLAT = _BATCH * _SEQ

_INFO = plsc.get_sparse_core_info()
_NC = _INFO.num_cores
_NS = _INFO.num_subcores
_NW = _NC * _NS
_ROWS_W = _FLAT // _NW
_CHUNK = 128
_NCHUNK = _ROWS_W // _CHUNK
_LANES = 16


def _emb_body(tok_hbm, table_hbm, pos_hbm, out_hbm, idx_v, trow_v, prow_v,
              out_v, sem):
    wid = lax.axis_index("s") * _NC + lax.axis_index("c")
    base = wid * _ROWS_W
    pltpu.sync_copy(tok_hbm.at[pl.ds(wid * _NCHUNK, _NCHUNK)], idx_v)
    copies = [
        pltpu.async_copy(table_hbm.at[idx_v.at[j]],
                         trow_v.at[pl.ds(j * _CHUNK, _CHUNK)], sem)
        for j in range(_NCHUNK)
    ]
    pbase = lax.rem(base, _SEQ)
    pltpu.sync_copy(pos_hbm.at[pl.ds(pbase, _ROWS_W)], prow_v)
    for cp in copies:
        cp.wait()

    iota = lax.iota(jnp.int32, _LANES)
    ones = jnp.full((_LANES,), 1, jnp.int32)

    def body(i, carry):
        tcol = ones * i
        for c in range(_EMBED // _LANES):
            sl = pl.ds(c * _LANES, _LANES)
            v = trow_v[i, sl] + prow_v[i, sl]
            plsc.store_scatter(out_v, [iota + c * _LANES, tcol], v)
        return carry

    lax.fori_loop(0, _ROWS_W, body, 0)
    bidx = base // _SEQ
    pltpu.sync_copy(out_v, out_hbm.at[bidx, :, pl.ds(pbase, _ROWS_W)])


_emb = functools.partial(
    pl.kernel,
    mesh=plsc.VectorSubcoreMesh(core_axis_name="c", subcore_axis_name="s"),
    out_type=jax.ShapeDtypeStruct((_BATCH, _EMBED, _SEQ), jnp.float32),
    scratch_types=[
        pltpu.VMEM((_NCHUNK, _CHUNK), jnp.int32),
        pltpu.VMEM((_ROWS_W, _EMBED), jnp.float32),
        pltpu.VMEM((_ROWS_W, _EMBED), jnp.float32),
        pltpu.VMEM((_EMBED, _ROWS_W), jnp.float32),
        pltpu.SemaphoreType.DMA,
    ],
    compiler_params=pltpu.CompilerParams(use_tc_tiling_on_sc=False,
                                         needs_layout_passes=False),
)(_emb_body)


def kernel(tokens, token_table, pos_table):
    tok = tokens.astype(jnp.int32).reshape(_FLAT // _CHUNK, _CHUNK)
    out = _emb(tok, token_table, pos_table)
    return jnp.transpose(out, (0, 2, 1)).astype(jnp.bfloat16)


# speedup vs baseline: 1.0936x; 1.0936x over previous
import functools

import jax
import jax.numpy as jnp
from jax import lax
from jax.experimental import pallas as pl
from jax.experimental.pallas import tpu as pltpu
from jax.experimental.pallas import tpu_sc as plsc

_BATCH = 4
_SEQ = 2048
_EMBED = 64
_FLAT = _BATCH * _SEQ

_INFO = plsc.get_sparse_core_info()
_NC = _INFO.num_cores
_NS = _INFO.num_subcores
_NW = _NC * _NS
_ROWS_W = _FLAT // _NW
_CHUNK = 128
_NCHUNK = _ROWS_W // _CHUNK
_LANES = 16


def _emb_body(tok_hbm, table_hbm, pos_hbm, out_hbm, idx_v, trow_v, prow_v,
              out_v, sem):
    wid = lax.axis_index("s") * _NC + lax.axis_index("c")
    base = wid * _ROWS_W
    pltpu.sync_copy(tok_hbm.at[pl.ds(wid * _NCHUNK, _NCHUNK)], idx_v)
    copies = [
        pltpu.async_copy(table_hbm.at[idx_v.at[j]],
                         trow_v.at[pl.ds(j * _CHUNK, _CHUNK)], sem)
        for j in range(_NCHUNK)
    ]
    pbase = lax.rem(base, _SEQ)
    pltpu.sync_copy(pos_hbm.at[pl.ds(pbase, _ROWS_W)], prow_v)
    for cp in copies:
        cp.wait()

    def body(i, carry):
        for c in range(_EMBED // _LANES):
            sl = pl.ds(c * _LANES, _LANES)
            out_v[i, sl] = trow_v[i, sl] + prow_v[i, sl]
        return carry

    lax.fori_loop(0, _ROWS_W, body, 0)
    pltpu.sync_copy(out_v, out_hbm.at[pl.ds(base, _ROWS_W)])


_emb = functools.partial(
    pl.kernel,
    mesh=plsc.VectorSubcoreMesh(core_axis_name="c", subcore_axis_name="s"),
    out_type=jax.ShapeDtypeStruct((_FLAT, _EMBED), jnp.float32),
    scratch_types=[
        pltpu.VMEM((_NCHUNK, _CHUNK), jnp.int32),
        pltpu.VMEM((_ROWS_W, _EMBED), jnp.float32),
        pltpu.VMEM((_ROWS_W, _EMBED), jnp.float32),
        pltpu.VMEM((_ROWS_W, _EMBED), jnp.float32),
        pltpu.SemaphoreType.DMA,
    ],
    compiler_params=pltpu.CompilerParams(use_tc_tiling_on_sc=False),
)(_emb_body)


def kernel(tokens, token_table, pos_table):
    tok = tokens.astype(jnp.int32).reshape(_FLAT // _CHUNK, _CHUNK)
    out = _emb(tok, token_table, pos_table)
    return out.reshape(_BATCH, _SEQ, _EMBED).astype(jnp.bfloat16)
